# baseline (speedup 1.0000x reference)
import functools

import jax
import jax.numpy as jnp
from jax import lax
from jax.experimental import pallas as pl
from jax.experimental.pallas import tpu as pltpu
from jax.experimental.pallas import tpu_sc as plsc

_VOCAB = 1000000
_D = 64
_B = 1024
_S = 20
_ROWS = 60
_SEG = _B * (1 + _S)
_NC = 2
_NS = 16
_NW = _NC * _NS
_PER_W = _SEG // _NW
_C = 2
_NCHUNK = _PER_W // _C
_NBUF = 4
_NSTEP = _NCHUNK // _NBUF
_NSLICE = _D // 16

_FB = 4096
_FMT_GRID = -(-_VOCAB // _FB)
_ROWS128 = _FMT_GRID * (_FB // 2)


def _tc_fmt_body(tabt_ref, out_ref):
    blkt = tabt_ref[...].T
    out_ref[:, 0:_D] = blkt[0 : _FB // 2]
    out_ref[:, _D:128] = blkt[_FB // 2 : _FB]


def _sc_body(idx_hbm, table_hbm, out_hbm, idx_v, rows_v, out_v, gsem0, gsem1, gsem2, gsem3):
    gsems = (gsem0, gsem1, gsem2, gsem3)
    wid = lax.axis_index("s") * _NC + lax.axis_index("c")
    base = wid * _PER_W

    pltpu.sync_copy(idx_hbm.at[wid], idx_v)

    def gather_start(j, b):
        pltpu.async_copy(table_hbm.at[idx_v.at[j]], rows_v.at[b], gsems[b])

    def gather_wait(b):
        pltpu.make_async_copy(
            table_hbm.at[idx_v.at[0]], rows_v.at[b], gsems[b]
        ).wait()

    for b in range(_NBUF):
        gather_start(b, b)

    def step(i, carry):
        for b in range(_NBUF):
            j = i * _NBUF + b
            gather_wait(b)
            for c in range(_C):
                accs = [
                    rows_v[b, c * _ROWS, pl.ds(16 * k, 16)]
                    for k in range(_NSLICE)
                ]
                for r in range(1, _ROWS):
                    for k in range(_NSLICE):
                        accs[k] = accs[k] + rows_v[
                            b, c * _ROWS + r, pl.ds(16 * k, 16)
                        ]
                seg = j * _C + c
                for k in range(_NSLICE):
                    out_v[seg, pl.ds(16 * k, 16)] = accs[k]
            nj = j + _NBUF

            @pl.when(nj < _NCHUNK)
            def _():
                gather_start(nj, b)

        return carry

    lax.fori_loop(0, _NSTEP, step, 0)
    pltpu.sync_copy(out_v, out_hbm.at[pl.ds(base, _PER_W)])


@jax.jit
def kernel(sub_index, derived_sub_indices, action_mask, table):
    mesh = plsc.VectorSubcoreMesh(core_axis_name="c", subcore_axis_name="s")

    tab128 = pl.pallas_call(
        _tc_fmt_body,
        grid=(_FMT_GRID,),
        in_specs=[pl.BlockSpec((_D, _FB), lambda c: (0, c))],
        out_specs=pl.BlockSpec((_FB // 2, 128), lambda c: (c, 0)),
        out_shape=jax.ShapeDtypeStruct((_ROWS128, 128), jnp.float32),
    )(table.T)
    tab_lin = tab128.reshape(2 * _ROWS128, _D)

    obs_idx = sub_index.reshape(_B, _ROWS).astype(jnp.int32)
    act_idx = derived_sub_indices.reshape(_B * _S, _ROWS).astype(jnp.int32)
    idx = jnp.concatenate([obs_idx, act_idx], axis=0)
    q = idx >> 11
    idx = ((q >> 1) << 12) + ((idx & 2047) << 1) + (q & 1)
    idx3 = idx.reshape(_NW, _NCHUNK, _C * _ROWS)

    kfn = functools.partial(
        pl.kernel,
        out_type=jax.ShapeDtypeStruct((_SEG, _D), jnp.float32),
        mesh=mesh,
        compiler_params=pltpu.CompilerParams(use_tc_tiling_on_sc=False),
        scratch_types=[
            pltpu.VMEM((_NCHUNK, _C * _ROWS), jnp.int32),
            pltpu.VMEM((_NBUF, _C * _ROWS, _D), jnp.float32),
            pltpu.VMEM((_PER_W, _D), jnp.float32),
            pltpu.SemaphoreType.DMA,
            pltpu.SemaphoreType.DMA,
            pltpu.SemaphoreType.DMA,
            pltpu.SemaphoreType.DMA,
        ],
    )(_sc_body)

    out = kfn(idx3, tab_lin)
    obs = out[:_B]
    action = out[_B:].reshape(_B, _S, _D)
    return (obs, action, action_mask)


# speedup vs baseline: 1.0903x; 1.0903x over previous
import functools

import jax
import jax.numpy as jnp
from jax import lax
from jax.experimental import pallas as pl
from jax.experimental.pallas import tpu as pltpu
from jax.experimental.pallas import tpu_sc as plsc

_VOCAB = 1000000
_D = 64
_B = 1024
_S = 20
_ROWS = 60
_SEG = _B * (1 + _S)
_NC = 2
_NS = 16
_NW = _NC * _NS
_PER_W = _SEG // _NW
_C = 2
_NCHUNK = _PER_W // _C
_NBUF = 2
_NSTEP = _NCHUNK // _NBUF
_NSLICE = _D // 16

_FB = 4096
_FMT_GRID = -(-_VOCAB // _FB)
_ROWS128 = _FMT_GRID * (_FB // 2)


def _tc_fmt_body(tabt_ref, out_ref):
    blkt = tabt_ref[...].T
    out_ref[:, 0:_D] = blkt[0 : _FB // 2]
    out_ref[:, _D:128] = blkt[_FB // 2 : _FB]


def _sc_body(idx_hbm, table_hbm, out_hbm, idx_v, rows_v, out_v, gsem0, gsem1):
    gsems = (gsem0, gsem1)
    wid = lax.axis_index("s") * _NC + lax.axis_index("c")
    base = wid * _PER_W

    pltpu.sync_copy(idx_hbm.at[wid], idx_v)

    def gather_start(j, b):
        pltpu.async_copy(table_hbm.at[idx_v.at[j]], rows_v.at[b], gsems[b])

    def gather_wait(b):
        pltpu.make_async_copy(
            table_hbm.at[idx_v.at[0]], rows_v.at[b], gsems[b]
        ).wait()

    for b in range(_NBUF):
        gather_start(b, b)

    def step(i, carry):
        for b in range(_NBUF):
            j = i * _NBUF + b
            gather_wait(b)
            for c in range(_C):
                acc0 = [
                    rows_v[b, c * _ROWS, pl.ds(16 * k, 16)]
                    for k in range(_NSLICE)
                ]
                acc1 = [
                    rows_v[b, c * _ROWS + 1, pl.ds(16 * k, 16)]
                    for k in range(_NSLICE)
                ]
                for r in range(2, _ROWS, 2):
                    for k in range(_NSLICE):
                        acc0[k] = acc0[k] + rows_v[
                            b, c * _ROWS + r, pl.ds(16 * k, 16)
                        ]
                        acc1[k] = acc1[k] + rows_v[
                            b, c * _ROWS + r + 1, pl.ds(16 * k, 16)
                        ]
                seg = j * _C + c
                for k in range(_NSLICE):
                    out_v[seg, pl.ds(16 * k, 16)] = acc0[k] + acc1[k]
            nj = j + _NBUF

            @pl.when(nj < _NCHUNK)
            def _():
                gather_start(nj, b)

        return carry

    lax.fori_loop(0, _NSTEP, step, 0)
    pltpu.sync_copy(out_v, out_hbm.at[pl.ds(base, _PER_W)])


@jax.jit
def kernel(sub_index, derived_sub_indices, action_mask, table):
    mesh = plsc.VectorSubcoreMesh(core_axis_name="c", subcore_axis_name="s")

    tab128 = pl.pallas_call(
        _tc_fmt_body,
        grid=(_FMT_GRID,),
        in_specs=[pl.BlockSpec((_D, _FB), lambda c: (0, c))],
        out_specs=pl.BlockSpec((_FB // 2, 128), lambda c: (c, 0)),
        out_shape=jax.ShapeDtypeStruct((_ROWS128, 128), jnp.float32),
    )(table.T)
    tab_lin = tab128.reshape(2 * _ROWS128, _D)

    obs_idx = sub_index.reshape(_B, _ROWS).astype(jnp.int32)
    act_idx = derived_sub_indices.reshape(_B * _S, _ROWS).astype(jnp.int32)
    idx = jnp.concatenate([obs_idx, act_idx], axis=0)
    q = idx >> 11
    idx = ((q >> 1) << 12) + ((idx & 2047) << 1) + (q & 1)
    idx3 = idx.reshape(_NW, _NCHUNK, _C * _ROWS)

    kfn = functools.partial(
        pl.kernel,
        out_type=jax.ShapeDtypeStruct((_SEG, _D), jnp.float32),
        mesh=mesh,
        compiler_params=pltpu.CompilerParams(use_tc_tiling_on_sc=False),
        scratch_types=[
            pltpu.VMEM((_NCHUNK, _C * _ROWS), jnp.int32),
            pltpu.VMEM((_NBUF, _C * _ROWS, _D), jnp.float32),
            pltpu.VMEM((_PER_W, _D), jnp.float32),
            pltpu.SemaphoreType.DMA,
            pltpu.SemaphoreType.DMA,
        ],
    )(_sc_body)

    out = kfn(idx3, tab_lin)
    obs = out[:_B]
    action = out[_B:].reshape(_B, _S, _D)
    return (obs, action, action_mask)


# speedup vs baseline: 1.1820x; 1.0841x over previous
import functools

import jax
import jax.numpy as jnp
from jax import lax
from jax.experimental import pallas as pl
from jax.experimental.pallas import tpu as pltpu
from jax.experimental.pallas import tpu_sc as plsc

_VOCAB = 1000000
_D = 64
_B = 1024
_S = 20
_ROWS = 60
_NC = 2
_NS = 16
_NW = _NC * _NS
_BPW = _B // _NW
_ACT_CHUNKS = _BPW * _S // 2
_OBS_CHUNKS = _BPW // 2

_FB = 4096
_FMT_GRID = -(-_VOCAB // _FB)
_ROWS128 = _FMT_GRID * (_FB // 2)


def _tc_fmt_body(tabt_ref, out_ref):
    blk = tabt_ref[...]
    eye = (
        lax.broadcasted_iota(jnp.int32, (_D, _D), 0)
        == lax.broadcasted_iota(jnp.int32, (_D, _D), 1)
    ).astype(jnp.float32)
    dn = (((0,), (0,)), ((), ()))
    lo = lax.dot_general(blk[:, 0 : _FB // 2], eye, dn,
                         preferred_element_type=jnp.float32)
    hi = lax.dot_general(blk[:, _FB // 2 : _FB], eye, dn,
                         preferred_element_type=jnp.float32)
    out_ref[:, 0:_D] = lo
    out_ref[:, _D:128] = hi


def _acc_rows(rows_ref, base, n, out_ref, orow):
    acc0 = [rows_ref[base, pl.ds(16 * k, 16)] for k in range(4)]
    acc1 = [rows_ref[base + 1, pl.ds(16 * k, 16)] for k in range(4)]
    for r in range(2, n, 2):
        for k in range(4):
            acc0[k] = acc0[k] + rows_ref[base + r, pl.ds(16 * k, 16)]
            acc1[k] = acc1[k] + rows_ref[base + r + 1, pl.ds(16 * k, 16)]
    for k in range(4):
        out_ref[orow, pl.ds(16 * k, 16)] = acc0[k] + acc1[k]


def _sc_body(obs_hbm, act_hbm, table_hbm, oout_hbm, aout_hbm,
             oidx_v, aidx_v, rows_v, oout_v, aout_v, gsem0, gsem1):
    gsems = (gsem0, gsem1)
    w = lax.axis_index("s") * _NC + lax.axis_index("c")
    b0 = w * _BPW

    pltpu.sync_copy(obs_hbm.at[pl.ds(b0, _BPW)], oidx_v)
    pltpu.sync_copy(act_hbm.at[pl.ds(b0, _BPW)], aidx_v)

    def ostart(t, b):
        pltpu.async_copy(
            table_hbm.at[oidx_v.at[t]], rows_v.at[b, pl.ds(0, _ROWS)], gsems[b])

    def owait(b):
        pltpu.make_async_copy(
            table_hbm.at[oidx_v.at[0]], rows_v.at[b, pl.ds(0, _ROWS)], gsems[b]
        ).wait()

    for b in range(2):
        ostart(b, b)

    def obs_step(i, carry):
        for b in range(2):
            t = i * 2 + b
            owait(b)
            _acc_rows(rows_v.at[b], 0, _ROWS, oout_v, t)
            nt = t + 2

            @pl.when(nt < _BPW)
            def _():
                ostart(nt, b)

        return carry

    lax.fori_loop(0, _OBS_CHUNKS, obs_step, 0)
    pltpu.sync_copy(oout_v, oout_hbm.at[pl.ds(b0, _BPW)])

    def astart(cj, b):
        jj = cj >> 5
        bl = cj & 31
        pltpu.async_copy(
            table_hbm.at[aidx_v.at[bl, pl.ds(jj * 120, 120)]],
            rows_v.at[b], gsems[b])

    def await_(b):
        pltpu.make_async_copy(
            table_hbm.at[aidx_v.at[0, pl.ds(0, 120)]], rows_v.at[b], gsems[b]
        ).wait()

    for b in range(2):
        astart(b, b)

    def act_step(i, carry):
        for b in range(2):
            cj = i * 2 + b
            await_(b)
            jj = cj >> 5
            bl = cj & 31
            r0 = bl * _S + jj * 2
            _acc_rows(rows_v.at[b], 0, _ROWS, aout_v, r0)
            _acc_rows(rows_v.at[b], _ROWS, _ROWS, aout_v, r0 + 1)
            ncj = cj + 2

            @pl.when(ncj < _ACT_CHUNKS)
            def _():
                astart(ncj, b)

        return carry

    lax.fori_loop(0, _ACT_CHUNKS // 2, act_step, 0)
    pltpu.sync_copy(aout_v, aout_hbm.at[pl.ds(b0 * _S, _BPW * _S)])


@jax.jit
def kernel(sub_index, derived_sub_indices, action_mask, table):
    mesh = plsc.VectorSubcoreMesh(core_axis_name="c", subcore_axis_name="s")

    tab128 = pl.pallas_call(
        _tc_fmt_body,
        grid=(_FMT_GRID,),
        in_specs=[pl.BlockSpec((_D, _FB), lambda c: (0, c))],
        out_specs=pl.BlockSpec((_FB // 2, 128), lambda c: (c, 0)),
        out_shape=jax.ShapeDtypeStruct((_ROWS128, 128), jnp.float32),
    )(table.T)
    tab_lin = tab128.reshape(2 * _ROWS128, _D)

    def remap(ix):
        ix = ix.astype(jnp.int32)
        q = ix >> 11
        return ((q >> 1) << 12) + ((ix & 2047) << 1) + (q & 1)

    obs2d = remap(sub_index.reshape(_B, _ROWS))
    act2d = remap(derived_sub_indices.reshape(_B, _S * _ROWS))

    kfn = functools.partial(
        pl.kernel,
        out_type=(
            jax.ShapeDtypeStruct((_B, _D), jnp.float32),
            jax.ShapeDtypeStruct((_B * _S, _D), jnp.float32),
        ),
        mesh=mesh,
        compiler_params=pltpu.CompilerParams(use_tc_tiling_on_sc=False),
        scratch_types=[
            pltpu.VMEM((_BPW, _ROWS), jnp.int32),
            pltpu.VMEM((_BPW, _S * _ROWS), jnp.int32),
            pltpu.VMEM((2, 2 * _ROWS, _D), jnp.float32),
            pltpu.VMEM((_BPW, _D), jnp.float32),
            pltpu.VMEM((_BPW * _S, _D), jnp.float32),
            pltpu.SemaphoreType.DMA,
            pltpu.SemaphoreType.DMA,
        ],
    )(_sc_body)

    obs, act = kfn(obs2d, act2d, tab_lin)
    action = act.reshape(_B, _S, _D)
    return (obs, action, action_mask)


# speedup vs baseline: 1.1872x; 1.0044x over previous
import functools

import jax
import jax.numpy as jnp
from jax import lax
from jax.experimental import pallas as pl
from jax.experimental.pallas import tpu as pltpu
from jax.experimental.pallas import tpu_sc as plsc

_VOCAB = 1000000
_D = 64
_B = 1024
_S = 20
_ROWS = 60
_NC = 2
_NS = 16
_NW = _NC * _NS
_BPW = _B // _NW
_ACT_CHUNKS = _BPW * _S // 2
_OBS_CHUNKS = _BPW // 2

_FB = 4096
_FMT_GRID = -(-_VOCAB // _FB)
_ROWS128 = _FMT_GRID * (_FB // 2)


def _tc_fmt_body(tabt_ref, out_ref):
    blkt = tabt_ref[...].T
    out_ref[:, 0:_D] = blkt[0 : _FB // 2]
    out_ref[:, _D:128] = blkt[_FB // 2 : _FB]


def _acc_rows(rows_ref, base, n, out_ref, orow):
    acc0 = [rows_ref[base, pl.ds(16 * k, 16)] for k in range(4)]
    acc1 = [rows_ref[base + 1, pl.ds(16 * k, 16)] for k in range(4)]
    for r in range(2, n, 2):
        for k in range(4):
            acc0[k] = acc0[k] + rows_ref[base + r, pl.ds(16 * k, 16)]
            acc1[k] = acc1[k] + rows_ref[base + r + 1, pl.ds(16 * k, 16)]
    for k in range(4):
        out_ref[orow, pl.ds(16 * k, 16)] = acc0[k] + acc1[k]


def _sc_body(obs_hbm, act_hbm, table_hbm, oout_hbm, aout_hbm,
             oidx_v, aidx_v, rows_v, oout_v, aout_v, gsem0, gsem1):
    gsems = (gsem0, gsem1)
    w = lax.axis_index("s") * _NC + lax.axis_index("c")
    b0 = w * _BPW

    pltpu.sync_copy(obs_hbm.at[pl.ds(b0, _BPW)], oidx_v)
    pltpu.sync_copy(act_hbm.at[pl.ds(b0, _BPW)], aidx_v)

    def ostart(t, b):
        pltpu.async_copy(
            table_hbm.at[oidx_v.at[t]], rows_v.at[b, pl.ds(0, _ROWS)], gsems[b])

    def owait(b):
        pltpu.make_async_copy(
            table_hbm.at[oidx_v.at[0]], rows_v.at[b, pl.ds(0, _ROWS)], gsems[b]
        ).wait()

    for b in range(2):
        ostart(b, b)

    def obs_step(i, carry):
        for b in range(2):
            t = i * 2 + b
            owait(b)
            _acc_rows(rows_v.at[b], 0, _ROWS, oout_v, t)
            nt = t + 2

            @pl.when(nt < _BPW)
            def _():
                ostart(nt, b)

        return carry

    lax.fori_loop(0, _OBS_CHUNKS, obs_step, 0)
    pltpu.sync_copy(oout_v, oout_hbm.at[pl.ds(b0, _BPW)])

    def astart(cj, b):
        jj = cj >> 5
        bl = cj & 31
        pltpu.async_copy(
            table_hbm.at[aidx_v.at[bl, pl.ds(jj * 120, 120)]],
            rows_v.at[b], gsems[b])

    def await_(b):
        pltpu.make_async_copy(
            table_hbm.at[aidx_v.at[0, pl.ds(0, 120)]], rows_v.at[b], gsems[b]
        ).wait()

    for b in range(2):
        astart(b, b)

    def act_step(i, carry):
        for b in range(2):
            cj = i * 2 + b
            await_(b)
            jj = cj >> 5
            bl = cj & 31
            r0 = bl * _S + jj * 2
            _acc_rows(rows_v.at[b], 0, _ROWS, aout_v, r0)
            _acc_rows(rows_v.at[b], _ROWS, _ROWS, aout_v, r0 + 1)
            ncj = cj + 2

            @pl.when(ncj < _ACT_CHUNKS)
            def _():
                astart(ncj, b)

        return carry

    lax.fori_loop(0, _ACT_CHUNKS // 2, act_step, 0)
    pltpu.sync_copy(aout_v, aout_hbm.at[pl.ds(b0 * _S, _BPW * _S)])


@jax.jit
def kernel(sub_index, derived_sub_indices, action_mask, table):
    mesh = plsc.VectorSubcoreMesh(core_axis_name="c", subcore_axis_name="s")

    tab128 = pl.pallas_call(
        _tc_fmt_body,
        grid=(_FMT_GRID,),
        in_specs=[pl.BlockSpec((_D, _FB), lambda c: (0, c))],
        out_specs=pl.BlockSpec((_FB // 2, 128), lambda c: (c, 0)),
        out_shape=jax.ShapeDtypeStruct((_ROWS128, 128), jnp.float32),
    )(table.T)
    tab_lin = tab128.reshape(2 * _ROWS128, _D)

    def remap(ix):
        ix = ix.astype(jnp.int32)
        q = ix >> 11
        return ((q >> 1) << 12) + ((ix & 2047) << 1) + (q & 1)

    obs2d = remap(sub_index.reshape(_B, _ROWS))
    act2d = remap(derived_sub_indices.reshape(_B, _S * _ROWS))

    kfn = functools.partial(
        pl.kernel,
        out_type=(
            jax.ShapeDtypeStruct((_B, _D), jnp.float32),
            jax.ShapeDtypeStruct((_B * _S, _D), jnp.float32),
        ),
        mesh=mesh,
        compiler_params=pltpu.CompilerParams(use_tc_tiling_on_sc=False),
        scratch_types=[
            pltpu.VMEM((_BPW, _ROWS), jnp.int32),
            pltpu.VMEM((_BPW, _S * _ROWS), jnp.int32),
            pltpu.VMEM((2, 2 * _ROWS, _D), jnp.float32),
            pltpu.VMEM((_BPW, _D), jnp.float32),
            pltpu.VMEM((_BPW * _S, _D), jnp.float32),
            pltpu.SemaphoreType.DMA,
            pltpu.SemaphoreType.DMA,
        ],
    )(_sc_body)

    obs, act = kfn(obs2d, act2d, tab_lin)
    action = act.reshape(_B, _S, _D)
    return (obs, action, action_mask)


# speedup vs baseline: 1.2964x; 1.0920x over previous
import functools

import jax
import jax.numpy as jnp
from jax import lax
from jax.experimental import pallas as pl
from jax.experimental.pallas import tpu as pltpu
from jax.experimental.pallas import tpu_sc as plsc

_VOCAB = 1000000
_D = 64
_B = 1024
_S = 20
_ROWS = 60
_NC = 2
_NS = 16
_NW = _NC * _NS
_BPW = _B // _NW
_ACT_CHUNKS = _BPW * _S // 2
_OBS_CHUNKS = _BPW // 2

_FB = 8192
_FMT_GRID = -(-_VOCAB // _FB)
_ROWS128 = _FMT_GRID * (_FB // 2)


def _tc_fmt_body(tabt_ref, out_ref):
    blkt = tabt_ref[...].T
    out_ref[:, 0:_D] = blkt[0 : _FB // 2]
    out_ref[:, _D:128] = blkt[_FB // 2 : _FB]


def _acc_rows(rows_ref, base, n, out_ref, orow):
    acc0 = [rows_ref[base, pl.ds(16 * k, 16)] for k in range(4)]
    acc1 = [rows_ref[base + 1, pl.ds(16 * k, 16)] for k in range(4)]
    for r in range(2, n, 2):
        for k in range(4):
            acc0[k] = acc0[k] + rows_ref[base + r, pl.ds(16 * k, 16)]
            acc1[k] = acc1[k] + rows_ref[base + r + 1, pl.ds(16 * k, 16)]
    for k in range(4):
        out_ref[orow, pl.ds(16 * k, 16)] = acc0[k] + acc1[k]


def _sc_body(obs_hbm, act_hbm, table_hbm, oout_hbm, aout_hbm,
             oidx_v, aidx_v, rows_v, oout_v, aout_v, gsem0, gsem1):
    gsems = (gsem0, gsem1)
    w = lax.axis_index("s") * _NC + lax.axis_index("c")
    b0 = w * _BPW

    pltpu.sync_copy(obs_hbm.at[pl.ds(w * (_BPW // 2), _BPW // 2)], oidx_v)
    pltpu.sync_copy(act_hbm.at[pl.ds(b0, _BPW)], aidx_v)

    def ostart(t, b):
        pltpu.async_copy(table_hbm.at[oidx_v.at[t]], rows_v.at[b], gsems[b])

    def owait(b):
        pltpu.make_async_copy(
            table_hbm.at[oidx_v.at[0]], rows_v.at[b], gsems[b]
        ).wait()

    for b in range(2):
        ostart(b, b)

    def obs_step(i, carry):
        for b in range(2):
            t = i * 2 + b
            owait(b)
            _acc_rows(rows_v.at[b], 0, _ROWS, oout_v, 2 * t)
            _acc_rows(rows_v.at[b], _ROWS, _ROWS, oout_v, 2 * t + 1)
            nt = t + 2

            @pl.when(nt < _BPW // 2)
            def _():
                ostart(nt, b)

        return carry

    lax.fori_loop(0, _OBS_CHUNKS // 2, obs_step, 0)
    pltpu.sync_copy(oout_v, oout_hbm.at[pl.ds(b0, _BPW)])

    def astart(cj, b):
        jj = cj >> 5
        bl = cj & 31
        pltpu.async_copy(
            table_hbm.at[aidx_v.at[bl, pl.ds(jj * 120, 120)]],
            rows_v.at[b], gsems[b])

    def await_(b):
        pltpu.make_async_copy(
            table_hbm.at[aidx_v.at[0, pl.ds(0, 120)]], rows_v.at[b], gsems[b]
        ).wait()

    for b in range(2):
        astart(b, b)

    def act_step(i, carry):
        for b in range(2):
            cj = i * 2 + b
            await_(b)
            jj = cj >> 5
            bl = cj & 31
            r0 = bl * _S + jj * 2
            _acc_rows(rows_v.at[b], 0, _ROWS, aout_v, r0)
            _acc_rows(rows_v.at[b], _ROWS, _ROWS, aout_v, r0 + 1)
            ncj = cj + 2

            @pl.when(ncj < _ACT_CHUNKS)
            def _():
                astart(ncj, b)

        return carry

    lax.fori_loop(0, _ACT_CHUNKS // 2, act_step, 0)
    pltpu.sync_copy(aout_v, aout_hbm.at[pl.ds(b0 * _S, _BPW * _S)])


@jax.jit
def kernel(sub_index, derived_sub_indices, action_mask, table):
    mesh = plsc.VectorSubcoreMesh(core_axis_name="c", subcore_axis_name="s")

    tab128 = pl.pallas_call(
        _tc_fmt_body,
        grid=(_FMT_GRID,),
        in_specs=[pl.BlockSpec((_D, _FB), lambda c: (0, c))],
        out_specs=pl.BlockSpec((_FB // 2, 128), lambda c: (c, 0)),
        out_shape=jax.ShapeDtypeStruct((_ROWS128, 128), jnp.float32),
    )(table.T)
    tab_lin = tab128.reshape(2 * _ROWS128, _D)

    def remap(ix):
        ix = ix.astype(jnp.int32)
        q = ix >> 12
        return ((q >> 1) << 13) + ((ix & 4095) << 1) + (q & 1)

    obs2d = remap(sub_index.reshape(_B // 2, 2 * _ROWS))
    act2d = remap(derived_sub_indices.reshape(_B, _S * _ROWS))

    kfn = functools.partial(
        pl.kernel,
        out_type=(
            jax.ShapeDtypeStruct((_B, _D), jnp.float32),
            jax.ShapeDtypeStruct((_B * _S, _D), jnp.float32),
        ),
        mesh=mesh,
        compiler_params=pltpu.CompilerParams(use_tc_tiling_on_sc=False),
        scratch_types=[
            pltpu.VMEM((_BPW // 2, 2 * _ROWS), jnp.int32),
            pltpu.VMEM((_BPW, _S * _ROWS), jnp.int32),
            pltpu.VMEM((2, 2 * _ROWS, _D), jnp.float32),
            pltpu.VMEM((_BPW, _D), jnp.float32),
            pltpu.VMEM((_BPW * _S, _D), jnp.float32),
            pltpu.SemaphoreType.DMA,
            pltpu.SemaphoreType.DMA,
        ],
    )(_sc_body)

    obs, act = kfn(obs2d, act2d, tab_lin)
    action = act.reshape(_B, _S, _D)
    return (obs, action, action_mask)


# speedup vs baseline: 1.3732x; 1.0593x over previous
import functools

import jax
import jax.numpy as jnp
from jax import lax
from jax.experimental import pallas as pl
from jax.experimental.pallas import tpu as pltpu
from jax.experimental.pallas import tpu_sc as plsc

_VOCAB = 1000000
_D = 64
_B = 1024
_S = 20
_ROWS = 60
_NC = 2
_NS = 16
_NW = _NC * _NS
_BPW = _B // _NW
_ACT_CHUNKS = _BPW * _S // 2
_OBS_CHUNKS = _BPW // 2

_FB = 16384
_FMT_GRID = -(-_VOCAB // _FB)
_ROWS128 = _FMT_GRID * (_FB // 2)


def _tc_fmt_body(tabt_ref, out_ref):
    blkt = tabt_ref[...].T
    out_ref[:, 0:_D] = blkt[0 : _FB // 2]
    out_ref[:, _D:128] = blkt[_FB // 2 : _FB]


def _acc_rows(rows_ref, base, n, out_ref, orow):
    acc0 = [rows_ref[base, pl.ds(16 * k, 16)] for k in range(4)]
    acc1 = [rows_ref[base + 1, pl.ds(16 * k, 16)] for k in range(4)]
    for r in range(2, n, 2):
        for k in range(4):
            acc0[k] = acc0[k] + rows_ref[base + r, pl.ds(16 * k, 16)]
            acc1[k] = acc1[k] + rows_ref[base + r + 1, pl.ds(16 * k, 16)]
    for k in range(4):
        out_ref[orow, pl.ds(16 * k, 16)] = acc0[k] + acc1[k]


def _sc_body(obs_hbm, act_hbm, table_hbm, oout_hbm, aout_hbm,
             oidx_v, aidx_v, rows_v, oout_v, aout_v, gsem0, gsem1):
    gsems = (gsem0, gsem1)
    w = lax.axis_index("s") * _NC + lax.axis_index("c")
    b0 = w * _BPW

    pltpu.sync_copy(obs_hbm.at[pl.ds(w * (_BPW // 2), _BPW // 2)], oidx_v)
    pltpu.sync_copy(act_hbm.at[pl.ds(b0, _BPW)], aidx_v)

    def ostart(t, b):
        pltpu.async_copy(table_hbm.at[oidx_v.at[t]], rows_v.at[b], gsems[b])

    def owait(b):
        pltpu.make_async_copy(
            table_hbm.at[oidx_v.at[0]], rows_v.at[b], gsems[b]
        ).wait()

    for b in range(2):
        ostart(b, b)

    def obs_step(i, carry):
        for b in range(2):
            t = i * 2 + b
            owait(b)
            _acc_rows(rows_v.at[b], 0, _ROWS, oout_v, 2 * t)
            _acc_rows(rows_v.at[b], _ROWS, _ROWS, oout_v, 2 * t + 1)
            nt = t + 2

            @pl.when(nt < _BPW // 2)
            def _():
                ostart(nt, b)

        return carry

    lax.fori_loop(0, _OBS_CHUNKS // 2, obs_step, 0)
    pltpu.sync_copy(oout_v, oout_hbm.at[pl.ds(b0, _BPW)])

    def astart(cj, b):
        jj = cj >> 5
        bl = cj & 31
        pltpu.async_copy(
            table_hbm.at[aidx_v.at[bl, pl.ds(jj * 120, 120)]],
            rows_v.at[b], gsems[b])

    def await_(b):
        pltpu.make_async_copy(
            table_hbm.at[aidx_v.at[0, pl.ds(0, 120)]], rows_v.at[b], gsems[b]
        ).wait()

    for b in range(2):
        astart(b, b)

    def act_step(i, carry):
        for b in range(2):
            cj = i * 2 + b
            await_(b)
            jj = cj >> 5
            bl = cj & 31
            r0 = bl * _S + jj * 2
            _acc_rows(rows_v.at[b], 0, _ROWS, aout_v, r0)
            _acc_rows(rows_v.at[b], _ROWS, _ROWS, aout_v, r0 + 1)
            ncj = cj + 2

            @pl.when(ncj < _ACT_CHUNKS)
            def _():
                astart(ncj, b)

        return carry

    lax.fori_loop(0, _ACT_CHUNKS // 2, act_step, 0)
    pltpu.sync_copy(aout_v, aout_hbm.at[pl.ds(b0 * _S, _BPW * _S)])


@jax.jit
def kernel(sub_index, derived_sub_indices, action_mask, table):
    mesh = plsc.VectorSubcoreMesh(core_axis_name="c", subcore_axis_name="s")

    tab128 = pl.pallas_call(
        _tc_fmt_body,
        grid=(_FMT_GRID,),
        in_specs=[pl.BlockSpec((_D, _FB), lambda c: (0, c))],
        out_specs=pl.BlockSpec((_FB // 2, 128), lambda c: (c, 0)),
        out_shape=jax.ShapeDtypeStruct((_ROWS128, 128), jnp.float32),
    )(table.T)
    tab_lin = tab128.reshape(2 * _ROWS128, _D)

    def remap(ix):
        ix = ix.astype(jnp.int32)
        q = ix >> 13
        return ((q >> 1) << 14) + ((ix & 8191) << 1) + (q & 1)

    obs2d = remap(sub_index.reshape(_B // 2, 2 * _ROWS))
    act2d = remap(derived_sub_indices.reshape(_B, _S * _ROWS))

    kfn = functools.partial(
        pl.kernel,
        out_type=(
            jax.ShapeDtypeStruct((_B, _D), jnp.float32),
            jax.ShapeDtypeStruct((_B * _S, _D), jnp.float32),
        ),
        mesh=mesh,
        compiler_params=pltpu.CompilerParams(use_tc_tiling_on_sc=False),
        scratch_types=[
            pltpu.VMEM((_BPW // 2, 2 * _ROWS), jnp.int32),
            pltpu.VMEM((_BPW, _S * _ROWS), jnp.int32),
            pltpu.VMEM((2, 2 * _ROWS, _D), jnp.float32),
            pltpu.VMEM((_BPW, _D), jnp.float32),
            pltpu.VMEM((_BPW * _S, _D), jnp.float32),
            pltpu.SemaphoreType.DMA,
            pltpu.SemaphoreType.DMA,
        ],
    )(_sc_body)

    obs, act = kfn(obs2d, act2d, tab_lin)
    action = act.reshape(_B, _S, _D)
    return (obs, action, action_mask)


# speedup vs baseline: 1.4067x; 1.0244x over previous
import functools

import jax
import jax.numpy as jnp
from jax import lax
from jax.experimental import pallas as pl
from jax.experimental.pallas import tpu as pltpu
from jax.experimental.pallas import tpu_sc as plsc

_VOCAB = 1000000
_D = 64
_B = 1024
_S = 20
_ROWS = 60
_NC = 2
_NS = 16
_NW = _NC * _NS
_BPW = _B // _NW
_ACT_CHUNKS = _BPW * _S // 2
_OBS_CHUNKS = _BPW // 2

_FB = 32768
_FMT_GRID = -(-_VOCAB // _FB)
_ROWS128 = _FMT_GRID * (_FB // 2)


def _tc_fmt_body(tabt_ref, out_ref):
    blkt = tabt_ref[...].T
    out_ref[:, 0:_D] = blkt[0 : _FB // 2]
    out_ref[:, _D:128] = blkt[_FB // 2 : _FB]


def _acc_rows(rows_ref, base, n, out_ref, orow):
    acc0 = [rows_ref[base, pl.ds(16 * k, 16)] for k in range(4)]
    acc1 = [rows_ref[base + 1, pl.ds(16 * k, 16)] for k in range(4)]
    for r in range(2, n, 2):
        for k in range(4):
            acc0[k] = acc0[k] + rows_ref[base + r, pl.ds(16 * k, 16)]
            acc1[k] = acc1[k] + rows_ref[base + r + 1, pl.ds(16 * k, 16)]
    for k in range(4):
        out_ref[orow, pl.ds(16 * k, 16)] = acc0[k] + acc1[k]


def _sc_body(obs_hbm, act_hbm, table_hbm, oout_hbm, aout_hbm,
             oidx_v, aidx_v, rows_v, oout_v, aout_v, gsem0, gsem1):
    gsems = (gsem0, gsem1)
    w = lax.axis_index("s") * _NC + lax.axis_index("c")
    b0 = w * _BPW

    pltpu.sync_copy(obs_hbm.at[pl.ds(w * (_BPW // 2), _BPW // 2)], oidx_v)
    pltpu.sync_copy(act_hbm.at[pl.ds(b0, _BPW)], aidx_v)

    def ostart(t, b):
        pltpu.async_copy(table_hbm.at[oidx_v.at[t]], rows_v.at[b], gsems[b])

    def owait(b):
        pltpu.make_async_copy(
            table_hbm.at[oidx_v.at[0]], rows_v.at[b], gsems[b]
        ).wait()

    for b in range(2):
        ostart(b, b)

    def obs_step(i, carry):
        for b in range(2):
            t = i * 2 + b
            owait(b)
            _acc_rows(rows_v.at[b], 0, _ROWS, oout_v, 2 * t)
            _acc_rows(rows_v.at[b], _ROWS, _ROWS, oout_v, 2 * t + 1)
            nt = t + 2

            @pl.when(nt < _BPW // 2)
            def _():
                ostart(nt, b)

        return carry

    lax.fori_loop(0, _OBS_CHUNKS // 2, obs_step, 0)
    pltpu.sync_copy(oout_v, oout_hbm.at[pl.ds(b0, _BPW)])

    def astart(cj, b):
        jj = cj >> 5
        bl = cj & 31
        pltpu.async_copy(
            table_hbm.at[aidx_v.at[bl, pl.ds(jj * 120, 120)]],
            rows_v.at[b], gsems[b])

    def await_(b):
        pltpu.make_async_copy(
            table_hbm.at[aidx_v.at[0, pl.ds(0, 120)]], rows_v.at[b], gsems[b]
        ).wait()

    for b in range(2):
        astart(b, b)

    def act_step(i, carry):
        for b in range(2):
            cj = i * 2 + b
            await_(b)
            jj = cj >> 5
            bl = cj & 31
            r0 = bl * _S + jj * 2
            _acc_rows(rows_v.at[b], 0, _ROWS, aout_v, r0)
            _acc_rows(rows_v.at[b], _ROWS, _ROWS, aout_v, r0 + 1)
            ncj = cj + 2

            @pl.when(ncj < _ACT_CHUNKS)
            def _():
                astart(ncj, b)

        return carry

    lax.fori_loop(0, _ACT_CHUNKS // 2, act_step, 0)
    pltpu.sync_copy(aout_v, aout_hbm.at[pl.ds(b0 * _S, _BPW * _S)])


@jax.jit
def kernel(sub_index, derived_sub_indices, action_mask, table):
    mesh = plsc.VectorSubcoreMesh(core_axis_name="c", subcore_axis_name="s")

    tab128 = pl.pallas_call(
        _tc_fmt_body,
        grid=(_FMT_GRID,),
        in_specs=[pl.BlockSpec((_D, _FB), lambda c: (0, c))],
        out_specs=pl.BlockSpec((_FB // 2, 128), lambda c: (c, 0)),
        out_shape=jax.ShapeDtypeStruct((_ROWS128, 128), jnp.float32),
    )(table.T)
    tab_lin = tab128.reshape(2 * _ROWS128, _D)

    def remap(ix):
        ix = ix.astype(jnp.int32)
        q = ix >> 14
        return ((q >> 1) << 15) + ((ix & 16383) << 1) + (q & 1)

    obs2d = remap(sub_index.reshape(_B // 2, 2 * _ROWS))
    act2d = remap(derived_sub_indices.reshape(_B, _S * _ROWS))

    kfn = functools.partial(
        pl.kernel,
        out_type=(
            jax.ShapeDtypeStruct((_B, _D), jnp.float32),
            jax.ShapeDtypeStruct((_B * _S, _D), jnp.float32),
        ),
        mesh=mesh,
        compiler_params=pltpu.CompilerParams(use_tc_tiling_on_sc=False),
        scratch_types=[
            pltpu.VMEM((_BPW // 2, 2 * _ROWS), jnp.int32),
            pltpu.VMEM((_BPW, _S * _ROWS), jnp.int32),
            pltpu.VMEM((2, 2 * _ROWS, _D), jnp.float32),
            pltpu.VMEM((_BPW, _D), jnp.float32),
            pltpu.VMEM((_BPW * _S, _D), jnp.float32),
            pltpu.SemaphoreType.DMA,
            pltpu.SemaphoreType.DMA,
        ],
    )(_sc_body)

    obs, act = kfn(obs2d, act2d, tab_lin)
    action = act.reshape(_B, _S, _D)
    return (obs, action, action_mask)
